# Initial kernel scaffold; baseline (speedup 1.0000x reference)
import jax
import jax.numpy as jnp
from jax.experimental import pallas as pl


def kernel(inputs, targets, positive_mask):
    raise NotImplementedError("write your pallas kernel here")


# speedup vs baseline: 1.8026x; 1.8026x over previous
import functools

import jax
import jax.numpy as jnp
from jax import lax
from jax.experimental import pallas as pl
from jax.experimental.pallas import tpu as pltpu
from jax.experimental.pallas import tpu_sc as plsc

_NUM_CLOTHES = 100000
_FEAT = 128
_BATCH = 1024
_SCALE = 16.0
_EPSILON = 0.1

_SC_CORES = 2
_SC_SUBCORES = 16
_NW = _SC_CORES * _SC_SUBCORES
_ROWS_PER_W = _BATCH // _NW
_CH = 128
_NCH = _BATCH // _CH


def _gather_positive_mask(pm_flat, targets):
    mesh = plsc.VectorSubcoreMesh(core_axis_name="c", subcore_axis_name="s")

    @functools.partial(
        pl.kernel,
        mesh=mesh,
        out_type=jax.ShapeDtypeStruct((_BATCH, _BATCH), jnp.float32),
        scratch_types=[
            pltpu.VMEM((_BATCH,), jnp.int32),
            pltpu.VMEM((_BATCH,), jnp.float32),
            pltpu.SemaphoreType.DMA,
        ],
    )
    def k(pm_hbm, tgt_hbm, out_hbm, idx_v, row_v, sem):
        wid = lax.axis_index("s") * _SC_CORES + lax.axis_index("c")
        base = wid * _ROWS_PER_W

        pltpu.sync_copy(tgt_hbm, idx_v)
        off0 = base * _NUM_CLOTHES

        def add_off(j, _):
            sl = pl.ds(j * 16, 16)
            idx_v[sl] = idx_v[sl] + off0
            return 0

        lax.fori_loop(0, _BATCH // 16, add_off, 0)

        def do_row(r, _):
            i = base + r
            copies = []
            for c in range(_NCH):
                sl = pl.ds(c * _CH, _CH)
                copies.append(
                    pltpu.async_copy(pm_hbm.at[idx_v.at[sl]], row_v.at[sl], sem)
                )
            for cp in copies:
                cp.wait()
            pltpu.sync_copy(row_v, out_hbm.at[i])

            def bump(j, _):
                sl = pl.ds(j * 16, 16)
                idx_v[sl] = idx_v[sl] + _NUM_CLOTHES
                return 0

            lax.fori_loop(0, _BATCH // 16, bump, 0)
            return 0

        lax.fori_loop(0, _ROWS_PER_W, do_row, 0)

    return k(pm_flat, targets)


def _bank_body(tcol_ref, trow_ref, x_ref, memn_ref):
    tcol = tcol_ref[...]
    trow = trow_ref[...]
    x = x_ref[...]
    same = (tcol == trow).astype(jnp.float32)
    ones = jnp.ones((_BATCH, 1), dtype=jnp.float32)
    cnt = lax.dot_general(same, ones, (((0,), (0,)), ((), ())),
                          preferred_element_type=jnp.float32)
    mem = lax.dot_general(same, x, (((0,), (0,)), ((), ())),
                          preferred_element_type=jnp.float32)
    mem = mem / cnt
    norm = jnp.sqrt(jnp.sum(mem * mem, axis=1, keepdims=True))
    memn_ref[...] = mem / jnp.maximum(norm, 1e-12)


def _build_bank(tcol, trow, x, bm=128):
    nb = _BATCH // bm
    return pl.pallas_call(
        _bank_body,
        grid=(nb,),
        in_specs=[
            pl.BlockSpec((_BATCH, 1), lambda b: (0, 0)),
            pl.BlockSpec((1, bm), lambda b: (0, b)),
            pl.BlockSpec((_BATCH, _FEAT), lambda b: (0, 0)),
        ],
        out_specs=pl.BlockSpec((bm, _FEAT), lambda b: (b, 0)),
        out_shape=jax.ShapeDtypeStruct((_BATCH, _FEAT), jnp.float32),
    )(tcol, trow, x)


def _loss_body(tcol_ref, trow_ref, x_ref, memn_ref, p_ref, out_ref, *, bm):
    pid = pl.program_id(0)
    tcol = tcol_ref[...]
    trow = trow_ref[...]
    x = x_ref[...]
    memn = memn_ref[...]
    P = p_ref[...]

    same = (tcol == trow).astype(jnp.float32)
    cnt = jnp.sum(same, axis=0, keepdims=True)
    w = 1.0 / cnt

    xn = x / jnp.maximum(jnp.sqrt(jnp.sum(x * x, axis=1, keepdims=True)), 1e-12)
    S = lax.dot_general(xn, memn, (((1,), (1,)), ((), ())),
                        preferred_element_type=jnp.float32) * _SCALE
    E = jnp.exp(S)
    neg = jnp.sum(w * (1.0 - P) * E, axis=1, keepdims=True)
    possum = jnp.sum(w * P, axis=1, keepdims=True)
    lp = S - jnp.log(neg + E)

    col = lax.broadcasted_iota(jnp.int32, (bm, _BATCH), 1)
    row = lax.broadcasted_iota(jnp.int32, (bm, _BATCH), 0)
    diag = jnp.sum(jnp.where(col == row + pid * bm, lp, 0.0), axis=1,
                   keepdims=True)
    pos_term = jnp.sum(w * P * lp, axis=1, keepdims=True)
    li = -(1.0 - _EPSILON) * diag - (_EPSILON / possum) * pos_term
    out_ref[...] = (jnp.sum(li) * (1.0 / _BATCH)).reshape(1, 1, 1)


def _compute_loss(tcol, trow, x, memn, P, bm=128):
    nb = _BATCH // bm
    parts = pl.pallas_call(
        functools.partial(_loss_body, bm=bm),
        grid=(nb,),
        in_specs=[
            pl.BlockSpec((_BATCH, 1), lambda i: (0, 0)),
            pl.BlockSpec((1, _BATCH), lambda i: (0, 0)),
            pl.BlockSpec((bm, _FEAT), lambda i: (i, 0)),
            pl.BlockSpec((_BATCH, _FEAT), lambda i: (0, 0)),
            pl.BlockSpec((bm, _BATCH), lambda i: (i, 0)),
        ],
        out_specs=pl.BlockSpec((1, 1, 1), lambda i: (i, 0, 0)),
        out_shape=jax.ShapeDtypeStruct((nb, 1, 1), jnp.float32),
    )(tcol, trow, x, memn, P)
    return parts


def kernel(inputs, targets, positive_mask):
    t32 = targets.astype(jnp.int32)
    tcol = t32.reshape(_BATCH, 1)
    trow = t32.reshape(1, _BATCH)
    pm_flat = positive_mask.reshape(-1)

    P = _gather_positive_mask(pm_flat, t32)
    memn = _build_bank(tcol, trow, inputs)
    parts = _compute_loss(tcol, trow, inputs, memn, P)
    return jnp.sum(parts).reshape(())


# speedup vs baseline: 24.0440x; 13.3383x over previous
import functools

import jax
import jax.numpy as jnp
from jax import lax
from jax.experimental import pallas as pl
from jax.experimental.pallas import tpu as pltpu
from jax.experimental.pallas import tpu_sc as plsc

_NUM_CLOTHES = 100000
_FEAT = 128
_BATCH = 1024
_SCALE = 16.0
_EPSILON = 0.1

_NUM_SLABS = _NUM_CLOTHES // 8

_SC_CORES = 2
_SC_SUBCORES = 16
_NW = _SC_CORES * _SC_SUBCORES
_TGT_PER_W = _BATCH // _NW
_SLAB_ROUND = 8
_N_ROUNDS = _TGT_PER_W // _SLAB_ROUND


def _gather_slabs(pm3, targets):
    mesh = plsc.VectorSubcoreMesh(core_axis_name="c", subcore_axis_name="s")

    @functools.partial(
        pl.kernel,
        mesh=mesh,
        out_type=jax.ShapeDtypeStruct((_BATCH, 8, _BATCH), jnp.float32),
        scratch_types=[
            pltpu.VMEM((_TGT_PER_W,), jnp.int32),
            pltpu.VMEM((_SLAB_ROUND, 8, _BATCH), jnp.float32),
            pltpu.SemaphoreType.DMA,
        ],
        compiler_params=pltpu.CompilerParams(use_tc_tiling_on_sc=True),
    )
    def k(pm3_hbm, tgt_hbm, out_hbm, idx_v, slab_v, sem):
        wid = lax.axis_index("s") * _SC_CORES + lax.axis_index("c")
        base = wid * _TGT_PER_W
        pltpu.sync_copy(tgt_hbm.at[pl.ds(base, _TGT_PER_W)], idx_v)
        for j in range(_TGT_PER_W // 16):
            sl = pl.ds(j * 16, 16)
            idx_v[sl] = lax.shift_right_logical(idx_v[sl], 3)

        def round_(g, _):
            cp = pltpu.async_copy(
                pm3_hbm.at[idx_v.at[pl.ds(g * _SLAB_ROUND, _SLAB_ROUND)]],
                slab_v,
                sem,
            )
            cp.wait()
            pltpu.sync_copy(
                slab_v, out_hbm.at[pl.ds(base + g * _SLAB_ROUND, _SLAB_ROUND)]
            )
            return 0

        lax.fori_loop(0, _N_ROUNDS, round_, 0)

    return k(pm3, targets)


def _bank_body(tcol_ref, trow_ref, x_ref, memn_ref):
    tcol = tcol_ref[...]
    trow = trow_ref[...]
    x = x_ref[...]
    same = (tcol == trow).astype(jnp.float32)
    ones = jnp.ones((_BATCH, 1), dtype=jnp.float32)
    cnt = lax.dot_general(same, ones, (((0,), (0,)), ((), ())),
                          preferred_element_type=jnp.float32)
    mem = lax.dot_general(same, x, (((0,), (0,)), ((), ())),
                          preferred_element_type=jnp.float32)
    mem = mem / cnt
    norm = jnp.sqrt(jnp.sum(mem * mem, axis=1, keepdims=True))
    memn_ref[...] = mem / jnp.maximum(norm, 1e-12)


def _build_bank(tcol, trow, x, bm=128):
    nb = _BATCH // bm
    return pl.pallas_call(
        _bank_body,
        grid=(nb,),
        in_specs=[
            pl.BlockSpec((_BATCH, 1), lambda b: (0, 0)),
            pl.BlockSpec((1, bm), lambda b: (0, b)),
            pl.BlockSpec((_BATCH, _FEAT), lambda b: (0, 0)),
        ],
        out_specs=pl.BlockSpec((bm, _FEAT), lambda b: (b, 0)),
        out_shape=jax.ShapeDtypeStruct((_BATCH, _FEAT), jnp.float32),
    )(tcol, trow, x)


def _loss_body(tcol_ref, trow_ref, x_ref, memn_ref, pts_ref, out_ref, *, bm):
    pid = pl.program_id(0)
    tcol = tcol_ref[...]
    trow = trow_ref[...]
    x = x_ref[...]
    memn = memn_ref[...]
    pts = pts_ref[...]

    same = (tcol == trow).astype(jnp.float32)
    cnt = jnp.sum(same, axis=1, keepdims=True)
    w = 1.0 / cnt

    tm = jnp.bitwise_and(tcol, 7)
    sel = lax.broadcasted_iota(jnp.int32, (_BATCH, 8, bm), 1) == tm[:, :, None]
    PT = jnp.sum(jnp.where(sel, pts, 0.0), axis=1)

    xn = x / jnp.maximum(jnp.sqrt(jnp.sum(x * x, axis=1, keepdims=True)), 1e-12)
    ST = lax.dot_general(memn, xn, (((1,), (1,)), ((), ())),
                         preferred_element_type=jnp.float32) * _SCALE
    ET = jnp.exp(ST)
    neg = jnp.sum(w * (1.0 - PT) * ET, axis=0, keepdims=True)
    possum = jnp.sum(w * PT, axis=0, keepdims=True)
    lpT = ST - jnp.log(neg + ET)

    ib = lax.broadcasted_iota(jnp.int32, (_BATCH, bm), 0)
    ii = lax.broadcasted_iota(jnp.int32, (_BATCH, bm), 1)
    diag = jnp.sum(jnp.where(ib == ii + pid * bm, lpT, 0.0), axis=0,
                   keepdims=True)
    pos_term = jnp.sum(w * PT * lpT, axis=0, keepdims=True)
    li = -(1.0 - _EPSILON) * diag - (_EPSILON / possum) * pos_term
    out_ref[...] = (jnp.sum(li) * (1.0 / _BATCH)).reshape(1, 1, 1)


def _compute_loss(tcol, trow, x, memn, pts, bm=128):
    nb = _BATCH // bm
    parts = pl.pallas_call(
        functools.partial(_loss_body, bm=bm),
        grid=(nb,),
        in_specs=[
            pl.BlockSpec((_BATCH, 1), lambda i: (0, 0)),
            pl.BlockSpec((1, _BATCH), lambda i: (0, 0)),
            pl.BlockSpec((bm, _FEAT), lambda i: (i, 0)),
            pl.BlockSpec((_BATCH, _FEAT), lambda i: (0, 0)),
            pl.BlockSpec((_BATCH, 8, bm), lambda i: (0, 0, i)),
        ],
        out_specs=pl.BlockSpec((1, 1, 1), lambda i: (i, 0, 0)),
        out_shape=jax.ShapeDtypeStruct((nb, 1, 1), jnp.float32),
    )(tcol, trow, x, memn, pts)
    return parts


def kernel(inputs, targets, positive_mask):
    t32 = targets.astype(jnp.int32)
    tcol = t32.reshape(_BATCH, 1)
    trow = t32.reshape(1, _BATCH)
    pm3 = positive_mask.T.reshape(_NUM_SLABS, 8, _BATCH)

    pts = _gather_slabs(pm3, t32)
    memn = _build_bank(tcol, trow, inputs)
    parts = _compute_loss(tcol, trow, inputs, memn, pts)
    return jnp.sum(parts).reshape(())


# speedup vs baseline: 24.3951x; 1.0146x over previous
import functools

import jax
import jax.numpy as jnp
from jax import lax
from jax.experimental import pallas as pl
from jax.experimental.pallas import tpu as pltpu
from jax.experimental.pallas import tpu_sc as plsc

_NUM_CLOTHES = 100000
_FEAT = 128
_BATCH = 1024
_SCALE = 16.0
_EPSILON = 0.1

_NUM_SLABS = _NUM_CLOTHES // 8

_SC_CORES = 2
_SC_SUBCORES = 16
_NW = _SC_CORES * _SC_SUBCORES
_TGT_PER_W = _BATCH // _NW
_SLAB_ROUND = 4
_N_ROUNDS = _TGT_PER_W // _SLAB_ROUND


def _gather_slabs(pm3, targets):
    mesh = plsc.VectorSubcoreMesh(core_axis_name="c", subcore_axis_name="s")

    @functools.partial(
        pl.kernel,
        mesh=mesh,
        out_type=jax.ShapeDtypeStruct((_BATCH, 8, _BATCH), jnp.float32),
        scratch_types=[
            pltpu.VMEM((_TGT_PER_W,), jnp.int32),
            pltpu.VMEM((8 * _N_ROUNDS,), jnp.int32),
            pltpu.VMEM((_SLAB_ROUND, 8, _BATCH), jnp.float32),
            pltpu.VMEM((_SLAB_ROUND, 8, _BATCH), jnp.float32),
            pltpu.SemaphoreType.DMA,
            pltpu.SemaphoreType.DMA,
        ],
        compiler_params=pltpu.CompilerParams(use_tc_tiling_on_sc=True,
                                             needs_layout_passes=False),
    )
    def k(pm3_hbm, tgt_hbm, out_hbm, tgt_v, idx_v, slab_a, slab_b, gsem, wsem):
        wid = lax.axis_index("s") * _SC_CORES + lax.axis_index("c")
        base = wid * _TGT_PER_W
        pltpu.sync_copy(tgt_hbm.at[pl.ds(base, _TGT_PER_W)], tgt_v)
        for j in range(_TGT_PER_W // 16):
            sl = pl.ds(j * 16, 16)
            m = lax.iota(jnp.int32, 16) + 16 * j
            pos = lax.shift_left(lax.shift_right_logical(m, 2), 3) + \
                jnp.bitwise_and(m, 3)
            plsc.store_scatter(idx_v, [pos],
                               lax.shift_right_logical(tgt_v[sl], 3))

        bufs = [slab_a, slab_b]

        def gather(g):
            return pltpu.async_copy(
                pm3_hbm.at[idx_v.at[pl.ds(g * 8, _SLAB_ROUND)]],
                bufs[g % 2],
                gsem,
            )

        gathers = {0: gather(0), 1: gather(1)}
        writes = {}
        for g in range(_N_ROUNDS):
            gathers[g].wait()
            writes[g] = pltpu.async_copy(
                bufs[g % 2],
                out_hbm.at[pl.ds(base + g * _SLAB_ROUND, _SLAB_ROUND)],
                wsem,
            )
            if g + 2 < _N_ROUNDS:
                writes[g].wait()
                gathers[g + 2] = gather(g + 2)
        for g in range(_N_ROUNDS):
            if g + 2 >= _N_ROUNDS:
                writes[g].wait()

    return k(pm3, targets)


def _bank_body(tcol_ref, trow_ref, x_ref, memn_ref):
    tcol = tcol_ref[...]
    trow = trow_ref[...]
    x = x_ref[...]
    same = (tcol == trow).astype(jnp.float32)
    ones = jnp.ones((_BATCH, 1), dtype=jnp.float32)
    cnt = lax.dot_general(same, ones, (((0,), (0,)), ((), ())),
                          preferred_element_type=jnp.float32)
    mem = lax.dot_general(same, x, (((0,), (0,)), ((), ())),
                          preferred_element_type=jnp.float32)
    mem = mem / cnt
    norm = jnp.sqrt(jnp.sum(mem * mem, axis=1, keepdims=True))
    memn_ref[...] = mem / jnp.maximum(norm, 1e-12)


def _build_bank(tcol, trow, x, bm=128):
    nb = _BATCH // bm
    return pl.pallas_call(
        _bank_body,
        grid=(nb,),
        in_specs=[
            pl.BlockSpec((_BATCH, 1), lambda b: (0, 0)),
            pl.BlockSpec((1, bm), lambda b: (0, b)),
            pl.BlockSpec((_BATCH, _FEAT), lambda b: (0, 0)),
        ],
        out_specs=pl.BlockSpec((bm, _FEAT), lambda b: (b, 0)),
        out_shape=jax.ShapeDtypeStruct((_BATCH, _FEAT), jnp.float32),
    )(tcol, trow, x)


def _loss_body(tcol_ref, trow_ref, x_ref, memn_ref, pts_ref, out_ref, *, bm):
    pid = pl.program_id(0)
    tcol = tcol_ref[...]
    trow = trow_ref[...]
    x = x_ref[...]
    memn = memn_ref[...]
    pts = pts_ref[...]

    same = (tcol == trow).astype(jnp.float32)
    cnt = jnp.sum(same, axis=1, keepdims=True)
    w = 1.0 / cnt

    tm = jnp.bitwise_and(tcol, 7)
    sel = lax.broadcasted_iota(jnp.int32, (_BATCH, 8, bm), 1) == tm[:, :, None]
    PT = jnp.sum(jnp.where(sel, pts, 0.0), axis=1)

    xn = x / jnp.maximum(jnp.sqrt(jnp.sum(x * x, axis=1, keepdims=True)), 1e-12)
    ST = lax.dot_general(memn, xn, (((1,), (1,)), ((), ())),
                         preferred_element_type=jnp.float32) * _SCALE
    ET = jnp.exp(ST)
    neg = jnp.sum(w * (1.0 - PT) * ET, axis=0, keepdims=True)
    possum = jnp.sum(w * PT, axis=0, keepdims=True)
    lpT = ST - jnp.log(neg + ET)

    ib = lax.broadcasted_iota(jnp.int32, (_BATCH, bm), 0)
    ii = lax.broadcasted_iota(jnp.int32, (_BATCH, bm), 1)
    diag = jnp.sum(jnp.where(ib == ii + pid * bm, lpT, 0.0), axis=0,
                   keepdims=True)
    pos_term = jnp.sum(w * PT * lpT, axis=0, keepdims=True)
    li = -(1.0 - _EPSILON) * diag - (_EPSILON / possum) * pos_term
    out_ref[...] = (jnp.sum(li) * (1.0 / _BATCH)).reshape(1, 1, 1)


def _compute_loss(tcol, trow, x, memn, pts, bm=128):
    nb = _BATCH // bm
    parts = pl.pallas_call(
        functools.partial(_loss_body, bm=bm),
        grid=(nb,),
        in_specs=[
            pl.BlockSpec((_BATCH, 1), lambda i: (0, 0)),
            pl.BlockSpec((1, _BATCH), lambda i: (0, 0)),
            pl.BlockSpec((bm, _FEAT), lambda i: (i, 0)),
            pl.BlockSpec((_BATCH, _FEAT), lambda i: (0, 0)),
            pl.BlockSpec((_BATCH, 8, bm), lambda i: (0, 0, i)),
        ],
        out_specs=pl.BlockSpec((1, 1, 1), lambda i: (i, 0, 0)),
        out_shape=jax.ShapeDtypeStruct((nb, 1, 1), jnp.float32),
    )(tcol, trow, x, memn, pts)
    return parts


def kernel(inputs, targets, positive_mask):
    t32 = targets.astype(jnp.int32)
    tcol = t32.reshape(_BATCH, 1)
    trow = t32.reshape(1, _BATCH)
    pm3 = positive_mask.T.reshape(_NUM_SLABS, 8, _BATCH)

    pts = _gather_slabs(pm3, t32)
    memn = _build_bank(tcol, trow, inputs)
    parts = _compute_loss(tcol, trow, inputs, memn, pts)
    return jnp.sum(parts).reshape(())


# speedup vs baseline: 34.6978x; 1.4223x over previous
import functools

import jax
import jax.numpy as jnp
from jax import lax
from jax.experimental import pallas as pl
from jax.experimental.pallas import tpu as pltpu
from jax.experimental.pallas import tpu_sc as plsc

_NUM_CLOTHES = 100000
_FEAT = 128
_BATCH = 1024
_SCALE = 16.0
_EPSILON = 0.1

_NUM_SLABS = _NUM_CLOTHES // 8

_SC_CORES = 2
_SC_SUBCORES = 16
_NW = _SC_CORES * _SC_SUBCORES
_TGT_PER_W = _BATCH // _NW
_SLAB_ROUND = 4
_N_ROUNDS = _TGT_PER_W // _SLAB_ROUND


def _gather_slabs(pm3, targets):
    mesh = plsc.VectorSubcoreMesh(core_axis_name="c", subcore_axis_name="s")

    @functools.partial(
        pl.kernel,
        mesh=mesh,
        out_type=jax.ShapeDtypeStruct((_BATCH, _BATCH), jnp.float32),
        scratch_types=[
            pltpu.VMEM((_TGT_PER_W,), jnp.int32),
            pltpu.VMEM((_TGT_PER_W,), jnp.int32),
            pltpu.VMEM((8 * _N_ROUNDS,), jnp.int32),
            pltpu.VMEM((_SLAB_ROUND, 8, _BATCH), jnp.float32),
            pltpu.VMEM((_SLAB_ROUND, 8, _BATCH), jnp.float32),
            pltpu.VMEM((_SLAB_ROUND, _BATCH), jnp.float32),
            pltpu.VMEM((_SLAB_ROUND, _BATCH), jnp.float32),
            pltpu.SemaphoreType.DMA,
            pltpu.SemaphoreType.DMA,
        ],
        compiler_params=pltpu.CompilerParams(use_tc_tiling_on_sc=True,
                                             needs_layout_passes=False),
    )
    def k(pm3_hbm, tgt_hbm, out_hbm, tgt_v, tmod_v, idx_v, slab_a, slab_b,
          row_a, row_b, gsem, wsem):
        wid = lax.axis_index("s") * _SC_CORES + lax.axis_index("c")
        base = wid * _TGT_PER_W
        pltpu.sync_copy(tgt_hbm.at[pl.ds(base, _TGT_PER_W)], tgt_v)
        for j in range(_TGT_PER_W // 16):
            sl = pl.ds(j * 16, 16)
            m = lax.iota(jnp.int32, 16) + 16 * j
            pos = lax.shift_left(lax.shift_right_logical(m, 2), 3) + \
                jnp.bitwise_and(m, 3)
            plsc.store_scatter(idx_v, [pos],
                               lax.shift_right_logical(tgt_v[sl], 3))
            tmod_v[sl] = jnp.bitwise_and(tgt_v[sl], 7)

        sbufs = [slab_a, slab_b]
        rbufs = [row_a, row_b]

        def gather(g):
            return pltpu.async_copy(
                pm3_hbm.at[idx_v.at[pl.ds(g * 8, _SLAB_ROUND)]],
                sbufs[g % 2],
                gsem,
            )

        def extract(g):
            slab = sbufs[g % 2]
            rows = rbufs[g % 2]
            lanes0 = lax.iota(jnp.int32, 16)
            for q in range(_SLAB_ROUND):
                qv = jnp.full((16,), q, dtype=jnp.int32)
                rv = plsc.load_gather(
                    tmod_v, [jnp.full((16,), g * _SLAB_ROUND + q, jnp.int32)]
                )
                def chunk(c, _):
                    lanes = lanes0 + c * 16
                    vals = plsc.load_gather(slab, [qv, rv, lanes])
                    rows[q, pl.ds(c * 16, 16)] = vals
                    return 0
                lax.fori_loop(0, _BATCH // 16, chunk, 0)

        gathers = {0: gather(0), 1: gather(1)}
        writes = {}
        for g in range(_N_ROUNDS):
            gathers[g].wait()
            if g - 2 >= 0:
                writes[g - 2].wait()
            extract(g)
            if g + 2 < _N_ROUNDS:
                gathers[g + 2] = gather(g + 2)
            writes[g] = pltpu.async_copy(
                rbufs[g % 2],
                out_hbm.at[pl.ds(base + g * _SLAB_ROUND, _SLAB_ROUND)],
                wsem,
            )
        writes[_N_ROUNDS - 2].wait()
        writes[_N_ROUNDS - 1].wait()

    return k(pm3, targets)


def _bank_body(tcol_ref, trow_ref, x_ref, memn_ref):
    tcol = tcol_ref[...]
    trow = trow_ref[...]
    x = x_ref[...]
    same = (tcol == trow).astype(jnp.float32)
    ones = jnp.ones((_BATCH, 1), dtype=jnp.float32)
    cnt = lax.dot_general(same, ones, (((0,), (0,)), ((), ())),
                          preferred_element_type=jnp.float32)
    mem = lax.dot_general(same, x, (((0,), (0,)), ((), ())),
                          preferred_element_type=jnp.float32)
    mem = mem / cnt
    norm = jnp.sqrt(jnp.sum(mem * mem, axis=1, keepdims=True))
    memn_ref[...] = mem / jnp.maximum(norm, 1e-12)


def _build_bank(tcol, trow, x, bm=128):
    nb = _BATCH // bm
    return pl.pallas_call(
        _bank_body,
        grid=(nb,),
        in_specs=[
            pl.BlockSpec((_BATCH, 1), lambda b: (0, 0)),
            pl.BlockSpec((1, bm), lambda b: (0, b)),
            pl.BlockSpec((_BATCH, _FEAT), lambda b: (0, 0)),
        ],
        out_specs=pl.BlockSpec((bm, _FEAT), lambda b: (b, 0)),
        out_shape=jax.ShapeDtypeStruct((_BATCH, _FEAT), jnp.float32),
    )(tcol, trow, x)


def _loss_body(tcol_ref, trow_ref, x_ref, memn_ref, pt_ref, out_ref, *, bm):
    pid = pl.program_id(0)
    tcol = tcol_ref[...]
    trow = trow_ref[...]
    x = x_ref[...]
    memn = memn_ref[...]
    PT = pt_ref[...]

    same = (tcol == trow).astype(jnp.float32)
    cnt = jnp.sum(same, axis=1, keepdims=True)
    w = 1.0 / cnt

    xn = x / jnp.maximum(jnp.sqrt(jnp.sum(x * x, axis=1, keepdims=True)), 1e-12)
    ST = lax.dot_general(memn, xn, (((1,), (1,)), ((), ())),
                         preferred_element_type=jnp.float32) * _SCALE
    ET = jnp.exp(ST)
    neg = jnp.sum(w * (1.0 - PT) * ET, axis=0, keepdims=True)
    possum = jnp.sum(w * PT, axis=0, keepdims=True)
    lpT = ST - jnp.log(neg + ET)

    ib = lax.broadcasted_iota(jnp.int32, (_BATCH, bm), 0)
    ii = lax.broadcasted_iota(jnp.int32, (_BATCH, bm), 1)
    diag = jnp.sum(jnp.where(ib == ii + pid * bm, lpT, 0.0), axis=0,
                   keepdims=True)
    pos_term = jnp.sum(w * PT * lpT, axis=0, keepdims=True)
    li = -(1.0 - _EPSILON) * diag - (_EPSILON / possum) * pos_term
    out_ref[...] = (jnp.sum(li) * (1.0 / _BATCH)).reshape(1, 1, 1)


def _compute_loss(tcol, trow, x, memn, pt, bm=128):
    nb = _BATCH // bm
    parts = pl.pallas_call(
        functools.partial(_loss_body, bm=bm),
        grid=(nb,),
        in_specs=[
            pl.BlockSpec((_BATCH, 1), lambda i: (0, 0)),
            pl.BlockSpec((1, _BATCH), lambda i: (0, 0)),
            pl.BlockSpec((bm, _FEAT), lambda i: (i, 0)),
            pl.BlockSpec((_BATCH, _FEAT), lambda i: (0, 0)),
            pl.BlockSpec((_BATCH, bm), lambda i: (0, i)),
        ],
        out_specs=pl.BlockSpec((1, 1, 1), lambda i: (i, 0, 0)),
        out_shape=jax.ShapeDtypeStruct((nb, 1, 1), jnp.float32),
    )(tcol, trow, x, memn, pt)
    return parts


def kernel(inputs, targets, positive_mask):
    t32 = targets.astype(jnp.int32)
    tcol = t32.reshape(_BATCH, 1)
    trow = t32.reshape(1, _BATCH)
    pm3 = positive_mask.T.reshape(_NUM_SLABS, 8, _BATCH)

    pts = _gather_slabs(pm3, t32)
    memn = _build_bank(tcol, trow, inputs)
    parts = _compute_loss(tcol, trow, inputs, memn, pts)
    return jnp.sum(parts).reshape(())
